# baseline (speedup 1.0000x reference)
import functools

import jax
import jax.numpy as jnp
from jax import lax
from jax.experimental import pallas as pl
from jax.experimental.pallas import tpu as pltpu
from jax.experimental.pallas import tpu_sc as plsc

N_NODES = 10000
N_EDGES = 320000
D = 128

NUM_CORES = 2
NUM_SUBCORES = 16
NW = NUM_CORES * NUM_SUBCORES
EDGES_PER_TILE = N_EDGES // NW
CHUNK = 80
NCHUNK = EDGES_PER_TILE // CHUNK
NBUF = 4
NSRC = 8
PRE = 5
OUT_TILES = 10
ROWS_PER_TILE = N_NODES // OUT_TILES
CNT_CHUNK = 200
CNT_PAD = 16 * ((CNT_CHUNK + 15) // 16)


def _sc_agg_body(with_counts, *refs):
  nsem = NSRC + 3 * NBUF + (NBUF if with_counts else 0)
  if with_counts:
    (x_hbm, ei_hbm, out_p, out_c, ones_v, zcnt, acc, cnt) = refs[:8]
    sv = refs[8:8 + NSRC]
    dv = refs[8 + NSRC:8 + NSRC + NBUF]
    rows = refs[8 + NSRC + NBUF:8 + NSRC + 2 * NBUF]
    sems = refs[8 + NSRC + 2 * NBUF:]
  else:
    (x_hbm, ei_hbm, out_p, acc) = refs[:4]
    ones_v = zcnt = cnt = None
    sv = refs[4:4 + NSRC]
    dv = refs[4 + NSRC:4 + NSRC + NBUF]
    rows = refs[4 + NSRC + NBUF:4 + NSRC + 2 * NBUF]
    sems = refs[4 + NSRC + 2 * NBUF:]
  isem = sems[:NSRC]
  jsem = sems[NSRC:NSRC + NBUF]
  gsem = sems[NSRC + NBUF:NSRC + 2 * NBUF]
  ssem = sems[NSRC + 2 * NBUF:NSRC + 3 * NBUF]
  csem = sems[NSRC + 3 * NBUF:] if with_counts else None

  cid = lax.axis_index("c")
  sid = lax.axis_index("s")
  wid = cid * NUM_SUBCORES + sid
  ebase = wid * EDGES_PER_TILE

  def ld_src(g, m):
    pltpu.async_copy(
        ei_hbm.at[pl.ds(ebase + g * CHUNK, CHUNK)], sv[m % NSRC],
        isem[m % NSRC])

  def src_wait(m):
    pltpu.make_async_copy(
        ei_hbm.at[pl.ds(0, CHUNK)], sv[m % NSRC], isem[m % NSRC]).wait()

  def ld_dst(g, m):
    pltpu.async_copy(
        ei_hbm.at[pl.ds(N_EDGES + ebase + g * CHUNK, CHUNK)], dv[m % NBUF],
        jsem[m % NBUF])

  def dst_wait(m):
    pltpu.make_async_copy(
        ei_hbm.at[pl.ds(0, CHUNK)], dv[m % NBUF], jsem[m % NBUF]).wait()

  def gat(m):
    pltpu.async_copy(x_hbm.at[sv[m % NSRC]], rows[m % NBUF], gsem[m % NBUF])

  def gat_wait(m):
    pltpu.make_async_copy(
        x_hbm.at[sv[m % NSRC]], rows[m % NBUF], gsem[m % NBUF]).wait()

  def scat(m):
    pltpu.async_copy(rows[m % NBUF], acc.at[dv[m % NBUF]], ssem[m % NBUF],
                     add=True)
    if with_counts:
      pltpu.async_copy(ones_v, cnt.at[dv[m % NBUF]], csem[m % NBUF], add=True)

  def scat_wait(m):
    pltpu.make_async_copy(
        rows[m % NBUF], acc.at[dv[m % NBUF]], ssem[m % NBUF]).wait()
    if with_counts:
      pltpu.make_async_copy(
          ones_v, cnt.at[dv[m % NBUF]], csem[m % NBUF]).wait()

  def step(g, m, first=False):
    gat_wait(m)
    dst_wait(m)
    scat(m)
    if not first:
      scat_wait(m - 1)

    @pl.when(g + 4 < NCHUNK)
    def _():
      ld_src(g + 4, m + 4)

    @pl.when(g + 3 < NCHUNK)
    def _():
      ld_dst(g + 3, m + 3)
      src_wait(m + 3)
      gat(m + 3)

  for g in range(4):
    ld_src(g, g)
  for g in range(3):
    ld_dst(g, g)
  for g in range(3):
    src_wait(g)
    gat(g)

  zbuf = rows[NBUF - 1]
  zeros16 = jnp.zeros((16,), jnp.float32)

  def _zrow(i, _):
    for j in range(D // 16):
      zbuf[i, pl.ds(j * 16, 16)] = zeros16
    return 0

  lax.fori_loop(0, CHUNK, _zrow, 0)

  @pl.when(sid < OUT_TILES)
  def _():
    for r in range(ROWS_PER_TILE // CHUNK):
      pltpu.sync_copy(
          zbuf, acc.at[pl.ds(sid * ROWS_PER_TILE + r * CHUNK, CHUNK), :])
    pltpu.sync_copy(
        zbuf.at[pl.ds(0, ROWS_PER_TILE % CHUNK), :],
        acc.at[pl.ds(sid * ROWS_PER_TILE + ROWS_PER_TILE - ROWS_PER_TILE % CHUNK,
                     ROWS_PER_TILE % CHUNK), :])

  if with_counts:
    ones16 = jnp.ones((16,), jnp.float32)
    for j in range(CHUNK // 16):
      ones_v[pl.ds(j * 16, 16)] = ones16

    def _zc(i, _):
      zcnt[pl.ds(i * 16, 16)] = zeros16
      return 0

    lax.fori_loop(0, CNT_PAD // 16, _zc, 0)

    @pl.when(sid < OUT_TILES)
    def _():
      for r in range(ROWS_PER_TILE // CNT_CHUNK):
        pltpu.sync_copy(
            zcnt.at[pl.ds(0, CNT_CHUNK)],
            cnt.at[pl.ds(sid * ROWS_PER_TILE + r * CNT_CHUNK, CNT_CHUNK)])

  plsc.subcore_barrier()

  step(0, 0, first=True)
  for g in range(1, PRE):
    step(g, g)

  def _oct(k, _):
    for j in range(8):
      step(8 * k + PRE + j, PRE + j)
    return 0

  lax.fori_loop(0, (NCHUNK - PRE) // 8, _oct, 0)
  scat_wait(NCHUNK - 1)

  plsc.subcore_barrier()

  @pl.when(sid < OUT_TILES)
  def _():
    pltpu.sync_copy(
        acc.at[pl.ds(sid * ROWS_PER_TILE, ROWS_PER_TILE), :],
        out_p.at[cid, pl.ds(sid * ROWS_PER_TILE, ROWS_PER_TILE), :],
    )
  if with_counts:
    @pl.when(sid < OUT_TILES)
    def _():
      for r in range(ROWS_PER_TILE // CNT_CHUNK):
        off = sid * ROWS_PER_TILE + r * CNT_CHUNK
        pltpu.sync_copy(cnt.at[pl.ds(off, CNT_CHUNK)],
                        zcnt.at[pl.ds(0, CNT_CHUNK)])
        pltpu.sync_copy(zcnt.at[pl.ds(0, CNT_CHUNK)],
                        out_c.at[pl.ds(cid * N_NODES + off, CNT_CHUNK)])


def _make_sc_agg(with_counts):
  mesh = plsc.VectorSubcoreMesh(
      core_axis_name="c", subcore_axis_name="s",
      num_cores=NUM_CORES, num_subcores=NUM_SUBCORES,
  )
  out_type = [jax.ShapeDtypeStruct((NUM_CORES, N_NODES, D), jnp.float32)]
  if with_counts:
    out_type.append(jax.ShapeDtypeStruct((NUM_CORES * N_NODES,), jnp.float32))
  scratch = []
  if with_counts:
    scratch += [
        pltpu.VMEM((CHUNK,), jnp.float32),
        pltpu.VMEM((CNT_PAD,), jnp.float32),
    ]
  scratch.append(pltpu.VMEM_SHARED((N_NODES, D), jnp.float32))
  if with_counts:
    scratch.append(pltpu.VMEM_SHARED((N_NODES,), jnp.float32))
  scratch += [pltpu.VMEM((CHUNK,), jnp.int32)] * NSRC
  scratch += [pltpu.VMEM((CHUNK,), jnp.int32)] * NBUF
  scratch += [pltpu.VMEM((CHUNK, D), jnp.float32)] * NBUF
  nsem = NSRC + 3 * NBUF + (NBUF if with_counts else 0)
  scratch += [pltpu.SemaphoreType.DMA] * nsem

  return pl.kernel(
      functools.partial(_sc_agg_body, with_counts),
      out_type=tuple(out_type) if with_counts else out_type[0],
      mesh=mesh,
      scratch_types=scratch,
  )


_sc_agg_with_counts = _make_sc_agg(True)
_sc_agg_no_counts = _make_sc_agg(False)

_DN_T = (((1,), (1,)), ((), ()))


def _mean(p_ref, c_ref):
  c = c_ref[0] + c_ref[1]
  inv = 1.0 / jnp.maximum(c, 1.0)
  return (p_ref[0] + p_ref[1]) * inv


def _tc_layer1_body(p_ref, c_ref, x_ref, wl1_ref, bl1_ref, wr1_ref, wr2_ref,
                    h_ref, xr2_ref):
  a = lax.dot_general(_mean(p_ref, c_ref), wl1_ref[...], _DN_T,
                      preferred_element_type=jnp.float32)
  a = a + lax.dot_general(x_ref[...], wr1_ref[...], _DN_T,
                          preferred_element_type=jnp.float32)
  h = jnp.maximum(a + bl1_ref[...], 0.0)
  h_ref[...] = h
  xr2_ref[...] = lax.dot_general(h, wr2_ref[...], _DN_T,
                                 preferred_element_type=jnp.float32)


def _tc_layer2_body(p_ref, c_ref, xr2_ref, wl2_ref, bl2_ref, o_ref):
  a = lax.dot_general(_mean(p_ref, c_ref), wl2_ref[...], _DN_T,
                      preferred_element_type=jnp.float32)
  o_ref[...] = a + xr2_ref[...] + bl2_ref[...]


_BLOCK = 1000
_NBLK = N_NODES // _BLOCK
_P_SPEC = pl.BlockSpec((NUM_CORES, _BLOCK, D), lambda i: (0, i, 0))
_C_SPEC = pl.BlockSpec((NUM_CORES, _BLOCK, 1), lambda i: (0, i, 0))
_V_SPEC = pl.BlockSpec((_BLOCK, D), lambda i: (i, 0))
_W_SPEC = pl.BlockSpec((D, D), lambda i: (0, 0))
_B_SPEC = pl.BlockSpec((1, D), lambda i: (0, 0))
_V_SHAPE = jax.ShapeDtypeStruct((N_NODES, D), jnp.float32)

_tc_layer1 = pl.pallas_call(
    _tc_layer1_body,
    grid=(_NBLK,),
    in_specs=[_P_SPEC, _C_SPEC, _V_SPEC, _W_SPEC, _B_SPEC, _W_SPEC, _W_SPEC],
    out_specs=(_V_SPEC, _V_SPEC),
    out_shape=(_V_SHAPE, _V_SHAPE),
)

_tc_layer2 = pl.pallas_call(
    _tc_layer2_body,
    grid=(_NBLK,),
    in_specs=[_P_SPEC, _C_SPEC, _V_SPEC, _W_SPEC, _B_SPEC],
    out_specs=_V_SPEC,
    out_shape=_V_SHAPE,
)


def kernel(x, edge_index, Wl1, bl1, Wr1, Wl2, bl2, Wr2):
  ei = edge_index.astype(jnp.int32).reshape(-1)

  p1, cnt = _sc_agg_with_counts(x, ei)
  cnt3 = cnt.reshape(NUM_CORES, N_NODES, 1)
  h, xr2 = _tc_layer1(p1, cnt3, x, Wl1, bl1.reshape(1, D), Wr1, Wr2)
  p2 = _sc_agg_no_counts(h, ei)
  out = _tc_layer2(p2, cnt3, xr2, Wl2, bl2.reshape(1, D))
  return out


# speedup vs baseline: 1.0050x; 1.0050x over previous
import functools

import jax
import jax.numpy as jnp
from jax import lax
from jax.experimental import pallas as pl
from jax.experimental.pallas import tpu as pltpu
from jax.experimental.pallas import tpu_sc as plsc

N_NODES = 10000
N_EDGES = 320000
D = 128

NUM_CORES = 2
NUM_SUBCORES = 16
NW = NUM_CORES * NUM_SUBCORES
EDGES_PER_TILE = N_EDGES // NW
CHUNK = 80
NCHUNK = EDGES_PER_TILE // CHUNK
NBUF = 4
NSRC = 8
PRE = 5
OUT_TILES = 10
ROWS_PER_TILE = N_NODES // OUT_TILES
CNT_CHUNK = 200
CNT_PAD = 16 * ((CNT_CHUNK + 15) // 16)


def _sc_agg_body(with_counts, *refs):
  nsem = NSRC + 3 * NBUF + (NBUF if with_counts else 0)
  if with_counts:
    (x_hbm, ei_hbm, out_p, out_c, ones_v, zcnt, acc, cnt) = refs[:8]
    sv = refs[8:8 + NSRC]
    dv = refs[8 + NSRC:8 + NSRC + NBUF]
    rows = refs[8 + NSRC + NBUF:8 + NSRC + 2 * NBUF]
    sems = refs[8 + NSRC + 2 * NBUF:]
  else:
    (x_hbm, ei_hbm, out_p, acc) = refs[:4]
    ones_v = zcnt = cnt = None
    sv = refs[4:4 + NSRC]
    dv = refs[4 + NSRC:4 + NSRC + NBUF]
    rows = refs[4 + NSRC + NBUF:4 + NSRC + 2 * NBUF]
    sems = refs[4 + NSRC + 2 * NBUF:]
  isem = sems[:NSRC]
  jsem = sems[NSRC:NSRC + NBUF]
  gsem = sems[NSRC + NBUF:NSRC + 2 * NBUF]
  ssem = sems[NSRC + 2 * NBUF:NSRC + 3 * NBUF]
  csem = sems[NSRC + 3 * NBUF:] if with_counts else None

  cid = lax.axis_index("c")
  sid = lax.axis_index("s")
  wid = cid * NUM_SUBCORES + sid
  ebase = wid * EDGES_PER_TILE

  def ld_src(g, m):
    pltpu.async_copy(
        ei_hbm.at[pl.ds(ebase + g * CHUNK, CHUNK)], sv[m % NSRC],
        isem[m % NSRC])

  def src_wait(m):
    pltpu.make_async_copy(
        ei_hbm.at[pl.ds(0, CHUNK)], sv[m % NSRC], isem[m % NSRC]).wait()

  def ld_dst(g, m):
    pltpu.async_copy(
        ei_hbm.at[pl.ds(N_EDGES + ebase + g * CHUNK, CHUNK)], dv[m % NBUF],
        jsem[m % NBUF])

  def dst_wait(m):
    pltpu.make_async_copy(
        ei_hbm.at[pl.ds(0, CHUNK)], dv[m % NBUF], jsem[m % NBUF]).wait()

  def gat(m):
    pltpu.async_copy(x_hbm.at[sv[m % NSRC]], rows[m % NBUF], gsem[m % NBUF])

  def gat_wait(m):
    pltpu.make_async_copy(
        x_hbm.at[sv[m % NSRC]], rows[m % NBUF], gsem[m % NBUF]).wait()

  def scat(m):
    pltpu.async_copy(rows[m % NBUF], acc.at[dv[m % NBUF]], ssem[m % NBUF],
                     add=True)
    if with_counts:
      pltpu.async_copy(ones_v, cnt.at[dv[m % NBUF]], csem[m % NBUF], add=True)

  def scat_wait(m):
    pltpu.make_async_copy(
        rows[m % NBUF], acc.at[dv[m % NBUF]], ssem[m % NBUF]).wait()
    if with_counts:
      pltpu.make_async_copy(
          ones_v, cnt.at[dv[m % NBUF]], csem[m % NBUF]).wait()

  def step(g, m, first=False):
    gat_wait(m)
    dst_wait(m)
    scat(m)
    if not first:
      scat_wait(m - 1)

    @pl.when(g + 4 < NCHUNK)
    def _():
      ld_src(g + 4, m + 4)

    @pl.when(g + 3 < NCHUNK)
    def _():
      ld_dst(g + 3, m + 3)
      src_wait(m + 3)
      gat(m + 3)

  for g in range(4):
    ld_src(g, g)
  for g in range(3):
    ld_dst(g, g)
  for g in range(3):
    src_wait(g)
    gat(g)

  zbuf = rows[NBUF - 1]
  zeros16 = jnp.zeros((16,), jnp.float32)

  def _zrow(i, _):
    for j in range(D // 16):
      zbuf[i, pl.ds(j * 16, 16)] = zeros16
    return 0

  lax.fori_loop(0, CHUNK, _zrow, 0)

  @pl.when(sid < OUT_TILES)
  def _():
    for r in range(ROWS_PER_TILE // CHUNK):
      pltpu.sync_copy(
          zbuf, acc.at[pl.ds(sid * ROWS_PER_TILE + r * CHUNK, CHUNK), :])
    pltpu.sync_copy(
        zbuf.at[pl.ds(0, ROWS_PER_TILE % CHUNK), :],
        acc.at[pl.ds(sid * ROWS_PER_TILE + ROWS_PER_TILE - ROWS_PER_TILE % CHUNK,
                     ROWS_PER_TILE % CHUNK), :])

  if with_counts:
    ones16 = jnp.ones((16,), jnp.float32)
    for j in range(CHUNK // 16):
      ones_v[pl.ds(j * 16, 16)] = ones16

    def _zc(i, _):
      zcnt[pl.ds(i * 16, 16)] = zeros16
      return 0

    lax.fori_loop(0, CNT_PAD // 16, _zc, 0)

    @pl.when(sid < OUT_TILES)
    def _():
      for r in range(ROWS_PER_TILE // CNT_CHUNK):
        pltpu.sync_copy(
            zcnt.at[pl.ds(0, CNT_CHUNK)],
            cnt.at[pl.ds(sid * ROWS_PER_TILE + r * CNT_CHUNK, CNT_CHUNK)])

  plsc.subcore_barrier()

  step(0, 0, first=True)
  for g in range(1, PRE):
    step(g, g)

  def _oct(k, _):
    for j in range(8):
      step(8 * k + PRE + j, PRE + j)
    return 0

  lax.fori_loop(0, (NCHUNK - PRE) // 8, _oct, 0)
  scat_wait(NCHUNK - 1)

  plsc.subcore_barrier()

  @pl.when(sid < OUT_TILES)
  def _():
    pltpu.sync_copy(
        acc.at[pl.ds(sid * ROWS_PER_TILE, ROWS_PER_TILE), :],
        out_p.at[cid, pl.ds(sid * ROWS_PER_TILE, ROWS_PER_TILE), :],
    )
  if with_counts:
    @pl.when(sid < OUT_TILES)
    def _():
      for r in range(ROWS_PER_TILE // CNT_CHUNK):
        off = sid * ROWS_PER_TILE + r * CNT_CHUNK
        pltpu.sync_copy(cnt.at[pl.ds(off, CNT_CHUNK)],
                        zcnt.at[pl.ds(0, CNT_CHUNK)])
        pltpu.sync_copy(zcnt.at[pl.ds(0, CNT_CHUNK)],
                        out_c.at[pl.ds(cid * N_NODES + off, CNT_CHUNK)])


def _make_sc_agg(with_counts):
  mesh = plsc.VectorSubcoreMesh(
      core_axis_name="c", subcore_axis_name="s",
      num_cores=NUM_CORES, num_subcores=NUM_SUBCORES,
  )
  out_type = [jax.ShapeDtypeStruct((NUM_CORES, N_NODES, D), jnp.float32)]
  if with_counts:
    out_type.append(jax.ShapeDtypeStruct((NUM_CORES * N_NODES,), jnp.float32))
  scratch = []
  if with_counts:
    scratch += [
        pltpu.VMEM((CHUNK,), jnp.float32),
        pltpu.VMEM((CNT_PAD,), jnp.float32),
    ]
  scratch.append(pltpu.VMEM_SHARED((N_NODES, D), jnp.float32))
  if with_counts:
    scratch.append(pltpu.VMEM_SHARED((N_NODES,), jnp.float32))
  scratch += [pltpu.VMEM((CHUNK,), jnp.int32)] * NSRC
  scratch += [pltpu.VMEM((CHUNK,), jnp.int32)] * NBUF
  scratch += [pltpu.VMEM((CHUNK, D), jnp.float32)] * NBUF
  nsem = NSRC + 3 * NBUF + (NBUF if with_counts else 0)
  scratch += [pltpu.SemaphoreType.DMA] * nsem

  return pl.kernel(
      functools.partial(_sc_agg_body, with_counts),
      out_type=tuple(out_type) if with_counts else out_type[0],
      mesh=mesh,
      scratch_types=scratch,
  )


_sc_agg_with_counts = _make_sc_agg(True)
_sc_agg_no_counts = _make_sc_agg(False)

_DN_T = (((1,), (1,)), ((), ()))


def _mean(p_ref, c_ref):
  c = c_ref[0] + c_ref[1]
  inv = 1.0 / jnp.maximum(c, 1.0)
  return (p_ref[0] + p_ref[1]) * inv


def _tc_mm_body(x_ref, w_ref, o_ref):
  o_ref[...] = lax.dot_general(x_ref[...], w_ref[...], _DN_T,
                               preferred_element_type=jnp.float32)


def _tc_combine_body(relu, p_ref, c_ref, xr_ref, wl_ref, bl_ref, o_ref):
  acc = lax.dot_general(_mean(p_ref, c_ref), wl_ref[...], _DN_T,
                        preferred_element_type=jnp.float32)
  acc = acc + xr_ref[...] + bl_ref[...]
  if relu:
    acc = jnp.maximum(acc, 0.0)
  o_ref[...] = acc


_BLOCK = 1000
_NBLK = N_NODES // _BLOCK
_P_SPEC = pl.BlockSpec((NUM_CORES, _BLOCK, D), lambda i: (0, i, 0))
_C_SPEC = pl.BlockSpec((NUM_CORES, _BLOCK, 1), lambda i: (0, i, 0))
_V_SPEC = pl.BlockSpec((_BLOCK, D), lambda i: (i, 0))
_W_SPEC = pl.BlockSpec((D, D), lambda i: (0, 0))
_B_SPEC = pl.BlockSpec((1, D), lambda i: (0, 0))
_V_SHAPE = jax.ShapeDtypeStruct((N_NODES, D), jnp.float32)

_tc_mm = pl.pallas_call(
    _tc_mm_body,
    grid=(_NBLK,),
    in_specs=[_V_SPEC, _W_SPEC],
    out_specs=_V_SPEC,
    out_shape=_V_SHAPE,
)


def _make_tc_combine(relu):
  return pl.pallas_call(
      functools.partial(_tc_combine_body, relu),
      grid=(_NBLK,),
      in_specs=[_P_SPEC, _C_SPEC, _V_SPEC, _W_SPEC, _B_SPEC],
      out_specs=_V_SPEC,
      out_shape=_V_SHAPE,
  )


_tc_combine_relu = _make_tc_combine(True)
_tc_combine_lin = _make_tc_combine(False)


def kernel(x, edge_index, Wl1, bl1, Wr1, Wl2, bl2, Wr2):
  ei = edge_index.astype(jnp.int32).reshape(-1)

  xr1 = _tc_mm(x, Wr1)
  p1, cnt = _sc_agg_with_counts(x, ei)
  cnt3 = cnt.reshape(NUM_CORES, N_NODES, 1)
  h = _tc_combine_relu(p1, cnt3, xr1, Wl1, bl1.reshape(1, D))
  xr2 = _tc_mm(h, Wr2)
  p2 = _sc_agg_no_counts(h, ei)
  out = _tc_combine_lin(p2, cnt3, xr2, Wl2, bl2.reshape(1, D))
  return out


# speedup vs baseline: 1.0222x; 1.0172x over previous
import functools

import jax
import jax.numpy as jnp
from jax import lax
from jax.experimental import pallas as pl
from jax.experimental.pallas import tpu as pltpu
from jax.experimental.pallas import tpu_sc as plsc

N_NODES = 10000
N_EDGES = 320000
D = 128

NUM_CORES = 2
NUM_SUBCORES = 16
NW = NUM_CORES * NUM_SUBCORES
EDGES_PER_TILE = N_EDGES // NW
CHUNK = 80
NCHUNK = EDGES_PER_TILE // CHUNK
NBUF = 4
NSRC = 8
PRE = 5
OUT_TILES = 10
ROWS_PER_TILE = N_NODES // OUT_TILES
CNT_CHUNK = 200
CNT_PAD = 16 * ((CNT_CHUNK + 15) // 16)


def _sc_agg_body(with_counts, *refs):
  nsem = NSRC + 3 * NBUF + (NBUF if with_counts else 0)
  if with_counts:
    (x_hbm, ei_hbm, out_p, out_c, ones_v, zcnt, acc, cnt) = refs[:8]
    sv = refs[8:8 + NSRC]
    dv = refs[8 + NSRC:8 + NSRC + NBUF]
    rows = refs[8 + NSRC + NBUF:8 + NSRC + 2 * NBUF]
    sems = refs[8 + NSRC + 2 * NBUF:]
  else:
    (x_hbm, ei_hbm, out_p, acc) = refs[:4]
    ones_v = zcnt = cnt = None
    sv = refs[4:4 + NSRC]
    dv = refs[4 + NSRC:4 + NSRC + NBUF]
    rows = refs[4 + NSRC + NBUF:4 + NSRC + 2 * NBUF]
    sems = refs[4 + NSRC + 2 * NBUF:]
  isem = sems[:NSRC]
  jsem = sems[NSRC:NSRC + NBUF]
  gsem = sems[NSRC + NBUF:NSRC + 2 * NBUF]
  ssem = sems[NSRC + 2 * NBUF:NSRC + 3 * NBUF]
  csem = sems[NSRC + 3 * NBUF:] if with_counts else None

  cid = lax.axis_index("c")
  sid = lax.axis_index("s")
  wid = cid * NUM_SUBCORES + sid
  ebase = wid * EDGES_PER_TILE

  def ld_src(g, m):
    pltpu.async_copy(
        ei_hbm.at[pl.ds(ebase + g * CHUNK, CHUNK)], sv[m % NSRC],
        isem[m % NSRC])

  def src_wait(m):
    pltpu.make_async_copy(
        ei_hbm.at[pl.ds(0, CHUNK)], sv[m % NSRC], isem[m % NSRC]).wait()

  def ld_dst(g, m):
    pltpu.async_copy(
        ei_hbm.at[pl.ds(N_EDGES + ebase + g * CHUNK, CHUNK)], dv[m % NBUF],
        jsem[m % NBUF])

  def dst_wait(m):
    pltpu.make_async_copy(
        ei_hbm.at[pl.ds(0, CHUNK)], dv[m % NBUF], jsem[m % NBUF]).wait()

  def gat(m):
    pltpu.async_copy(x_hbm.at[sv[m % NSRC]], rows[m % NBUF], gsem[m % NBUF])

  def gat_wait(m):
    pltpu.make_async_copy(
        x_hbm.at[sv[m % NSRC]], rows[m % NBUF], gsem[m % NBUF]).wait()

  def scat(m):
    pltpu.async_copy(rows[m % NBUF], acc.at[dv[m % NBUF]], ssem[m % NBUF],
                     add=True)
    if with_counts:
      pltpu.async_copy(ones_v, cnt.at[dv[m % NBUF]], csem[m % NBUF], add=True)

  def scat_wait(m):
    pltpu.make_async_copy(
        rows[m % NBUF], acc.at[dv[m % NBUF]], ssem[m % NBUF]).wait()
    if with_counts:
      pltpu.make_async_copy(
          ones_v, cnt.at[dv[m % NBUF]], csem[m % NBUF]).wait()

  def step(g, m, first=False):
    gat_wait(m)
    dst_wait(m)
    scat(m)
    if not first:
      scat_wait(m - 1)

    @pl.when(g + 4 < NCHUNK)
    def _():
      ld_src(g + 4, m + 4)

    @pl.when(g + 3 < NCHUNK)
    def _():
      ld_dst(g + 3, m + 3)
      src_wait(m + 3)
      gat(m + 3)

  for g in range(4):
    ld_src(g, g)
  for g in range(3):
    ld_dst(g, g)
  for g in range(3):
    src_wait(g)
    gat(g)

  zbuf = rows[NBUF - 1]
  zeros16 = jnp.zeros((16,), jnp.float32)

  def _zrow(i, _):
    for j in range(D // 16):
      zbuf[i, pl.ds(j * 16, 16)] = zeros16
    return 0

  lax.fori_loop(0, CHUNK, _zrow, 0)

  @pl.when(sid < OUT_TILES)
  def _():
    for r in range(ROWS_PER_TILE // CHUNK):
      pltpu.sync_copy(
          zbuf, acc.at[pl.ds(sid * ROWS_PER_TILE + r * CHUNK, CHUNK), :])
    pltpu.sync_copy(
        zbuf.at[pl.ds(0, ROWS_PER_TILE % CHUNK), :],
        acc.at[pl.ds(sid * ROWS_PER_TILE + ROWS_PER_TILE - ROWS_PER_TILE % CHUNK,
                     ROWS_PER_TILE % CHUNK), :])

  if with_counts:
    ones16 = jnp.ones((16,), jnp.float32)
    for j in range(CHUNK // 16):
      ones_v[pl.ds(j * 16, 16)] = ones16

    def _zc(i, _):
      zcnt[pl.ds(i * 16, 16)] = zeros16
      return 0

    lax.fori_loop(0, CNT_PAD // 16, _zc, 0)

    @pl.when(sid < OUT_TILES)
    def _():
      for r in range(ROWS_PER_TILE // CNT_CHUNK):
        pltpu.sync_copy(
            zcnt.at[pl.ds(0, CNT_CHUNK)],
            cnt.at[pl.ds(sid * ROWS_PER_TILE + r * CNT_CHUNK, CNT_CHUNK)])

  plsc.subcore_barrier()

  step(0, 0, first=True)
  for g in range(1, PRE):
    step(g, g)

  def _oct(k, _):
    for j in range(8):
      step(8 * k + PRE + j, PRE + j)
    return 0

  lax.fori_loop(0, (NCHUNK - PRE) // 8, _oct, 0)
  scat_wait(NCHUNK - 1)

  plsc.subcore_barrier()

  @pl.when(sid < OUT_TILES)
  def _():
    pltpu.sync_copy(
        acc.at[pl.ds(sid * ROWS_PER_TILE, ROWS_PER_TILE), :],
        out_p.at[cid, pl.ds(sid * ROWS_PER_TILE, ROWS_PER_TILE), :],
    )
  if with_counts:
    @pl.when(sid < OUT_TILES)
    def _():
      for r in range(ROWS_PER_TILE // CNT_CHUNK):
        off = sid * ROWS_PER_TILE + r * CNT_CHUNK
        pltpu.sync_copy(cnt.at[pl.ds(off, CNT_CHUNK)],
                        zcnt.at[pl.ds(0, CNT_CHUNK)])
        pltpu.sync_copy(zcnt.at[pl.ds(0, CNT_CHUNK)],
                        out_c.at[pl.ds(cid * N_NODES + off, CNT_CHUNK)])


def _make_sc_agg(with_counts):
  mesh = plsc.VectorSubcoreMesh(
      core_axis_name="c", subcore_axis_name="s",
      num_cores=NUM_CORES, num_subcores=NUM_SUBCORES,
  )
  out_type = [jax.ShapeDtypeStruct((NUM_CORES, N_NODES, D), jnp.float32)]
  if with_counts:
    out_type.append(jax.ShapeDtypeStruct((NUM_CORES * N_NODES,), jnp.float32))
  scratch = []
  if with_counts:
    scratch += [
        pltpu.VMEM((CHUNK,), jnp.float32),
        pltpu.VMEM((CNT_PAD,), jnp.float32),
    ]
  scratch.append(pltpu.VMEM_SHARED((N_NODES, D), jnp.float32))
  if with_counts:
    scratch.append(pltpu.VMEM_SHARED((N_NODES,), jnp.float32))
  scratch += [pltpu.VMEM((CHUNK,), jnp.int32)] * NSRC
  scratch += [pltpu.VMEM((CHUNK,), jnp.int32)] * NBUF
  scratch += [pltpu.VMEM((CHUNK, D), jnp.float32)] * NBUF
  nsem = NSRC + 3 * NBUF + (NBUF if with_counts else 0)
  scratch += [pltpu.SemaphoreType.DMA] * nsem

  return pl.kernel(
      functools.partial(_sc_agg_body, with_counts),
      out_type=tuple(out_type) if with_counts else out_type[0],
      mesh=mesh,
      scratch_types=scratch,
  )


_sc_agg_with_counts = _make_sc_agg(True)
_sc_agg_no_counts = _make_sc_agg(False)

_DN_T = (((1,), (1,)), ((), ()))


def _mean(p_ref, c_ref):
  c = c_ref[0] + c_ref[1]
  inv = 1.0 / jnp.maximum(c, 1.0)
  return (p_ref[0] + p_ref[1]) * inv


def _tc_mm_body(x_ref, w_ref, o_ref):
  o_ref[...] = lax.dot_general(x_ref[...], w_ref[...], _DN_T,
                               preferred_element_type=jnp.float32)


def _tc_combine_body(relu, p_ref, c_ref, xr_ref, wl_ref, bl_ref, o_ref):
  acc = lax.dot_general(_mean(p_ref, c_ref), wl_ref[...], _DN_T,
                        preferred_element_type=jnp.float32)
  acc = acc + xr_ref[...] + bl_ref[...]
  if relu:
    acc = jnp.maximum(acc, 0.0)
  o_ref[...] = acc


_BLOCK = 2000
_NBLK = N_NODES // _BLOCK
_P_SPEC = pl.BlockSpec((NUM_CORES, _BLOCK, D), lambda i: (0, i, 0))
_C_SPEC = pl.BlockSpec((NUM_CORES, _BLOCK, 1), lambda i: (0, i, 0))
_V_SPEC = pl.BlockSpec((_BLOCK, D), lambda i: (i, 0))
_W_SPEC = pl.BlockSpec((D, D), lambda i: (0, 0))
_B_SPEC = pl.BlockSpec((1, D), lambda i: (0, 0))
_V_SHAPE = jax.ShapeDtypeStruct((N_NODES, D), jnp.float32)

_tc_mm = pl.pallas_call(
    _tc_mm_body,
    grid=(_NBLK,),
    in_specs=[_V_SPEC, _W_SPEC],
    out_specs=_V_SPEC,
    out_shape=_V_SHAPE,
)


def _make_tc_combine(relu):
  return pl.pallas_call(
      functools.partial(_tc_combine_body, relu),
      grid=(_NBLK,),
      in_specs=[_P_SPEC, _C_SPEC, _V_SPEC, _W_SPEC, _B_SPEC],
      out_specs=_V_SPEC,
      out_shape=_V_SHAPE,
  )


_tc_combine_relu = _make_tc_combine(True)
_tc_combine_lin = _make_tc_combine(False)


def kernel(x, edge_index, Wl1, bl1, Wr1, Wl2, bl2, Wr2):
  ei = edge_index.astype(jnp.int32).reshape(-1)

  xr1 = _tc_mm(x, Wr1)
  p1, cnt = _sc_agg_with_counts(x, ei)
  cnt3 = cnt.reshape(NUM_CORES, N_NODES, 1)
  h = _tc_combine_relu(p1, cnt3, xr1, Wl1, bl1.reshape(1, D))
  xr2 = _tc_mm(h, Wr2)
  p2 = _sc_agg_no_counts(h, ei)
  out = _tc_combine_lin(p2, cnt3, xr2, Wl2, bl2.reshape(1, D))
  return out
